# Initial kernel scaffold; baseline (speedup 1.0000x reference)
import jax
import jax.numpy as jnp
from jax.experimental import pallas as pl


def kernel(x, edge_index, W1l, W1r, b1, W2l, W2r, b2, W3l, W3r, b3, W4l, W4r, b4):
    raise NotImplementedError("write your pallas kernel here")


# speedup vs baseline: 4.1971x; 4.1971x over previous
import functools

import jax
import jax.numpy as jnp
from jax import lax
from jax.experimental import pallas as pl
from jax.experimental.pallas import tpu as pltpu
from jax.experimental.pallas import tpu_sc as plsc

N = 10000
D = 128
E = 320000
CHUNK = 128
NC, NS = 2, 16
NW = NC * NS
CW = 79
EPAD = NW * CW * CHUNK
NPAD = 10112
RPS = NPAD // NS
CNTW = 16
ROWS_BLK = 2000

@functools.cache
def _make_sc_agg(first: bool):
  mesh = plsc.VectorSubcoreMesh(
      core_axis_name="core", subcore_axis_name="subcore",
      num_cores=NC, num_subcores=NS)
  out_type = [jax.ShapeDtypeStruct((NC, NPAD, D), jnp.float32)]
  scratch = [
      pltpu.VMEM_SHARED((NPAD, D), jnp.float32),
      pltpu.VMEM((2, CHUNK), jnp.int32),
      pltpu.VMEM((CHUNK, D), jnp.float32),
  ]
  if first:
    out_type.append(jax.ShapeDtypeStruct((NC, NPAD, CNTW), jnp.float32))
    scratch += [
        pltpu.VMEM_SHARED((NPAD, CNTW), jnp.float32),
        pltpu.VMEM((CHUNK, CNTW), jnp.float32),
    ]

  def body(h_hbm, src_hbm, dst_hbm, z128_hbm, *rest):
    if first:
      (z16_hbm, ones_hbm, agg_out, cnt_out,
       agg_sh, idx_v, rows_v, cnt_sh, ones_v) = rest
    else:
      agg_out, agg_sh, idx_v, rows_v = rest
    c = lax.axis_index("core")
    s = lax.axis_index("subcore")
    w = c * NS + s
    r0 = s * RPS
    pltpu.sync_copy(z128_hbm.at[pl.ds(r0, RPS)], agg_sh.at[pl.ds(r0, RPS)])
    if first:
      pltpu.sync_copy(z16_hbm.at[pl.ds(r0, RPS)], cnt_sh.at[pl.ds(r0, RPS)])
      pltpu.sync_copy(ones_hbm, ones_v)
    plsc.subcore_barrier()

    @pl.loop(0, CW)
    def _(i):
      base = (w * CW + i) * CHUNK
      pltpu.sync_copy(src_hbm.at[pl.ds(base, CHUNK)], idx_v.at[0])
      pltpu.sync_copy(dst_hbm.at[pl.ds(base, CHUNK)], idx_v.at[1])
      pltpu.sync_copy(h_hbm.at[idx_v.at[0]], rows_v)
      pltpu.sync_copy(rows_v, agg_sh.at[idx_v.at[1]], add=True)
      if first:
        pltpu.sync_copy(ones_v, cnt_sh.at[idx_v.at[1]], add=True)

    plsc.subcore_barrier()
    pltpu.sync_copy(agg_sh.at[pl.ds(r0, RPS)], agg_out.at[c, pl.ds(r0, RPS)])
    if first:
      pltpu.sync_copy(cnt_sh.at[pl.ds(r0, RPS)], cnt_out.at[c, pl.ds(r0, RPS)])

  return pl.kernel(body, out_type=tuple(out_type) if first else out_type[0],
                   mesh=mesh, scratch_types=scratch,
                   compiler_params=pltpu.CompilerParams(
                       use_tc_tiling_on_sc=False))


def _elu(z):
  return jnp.where(z > 0, z, jnp.exp(z) - 1.0)


def _tc_first_body(agg0, agg1, cnt0, cnt1, h, wl, wr, b, out, invout):
  cnt = cnt0[0][:, 0:1] + cnt1[0][:, 0:1]
  inv = 1.0 / jnp.maximum(cnt, 1.0)
  invout[...] = jnp.broadcast_to(inv, (ROWS_BLK, D))
  mean = (agg0[0] + agg1[0]) * inv
  z = (jnp.dot(mean, wl[...], preferred_element_type=jnp.float32,
               precision=lax.Precision.HIGHEST)
       + jnp.dot(h[...], wr[...], preferred_element_type=jnp.float32,
                 precision=lax.Precision.HIGHEST) + b[...])
  out[...] = _elu(z)


def _tc_mid_body(last, agg0, agg1, inv, h, wl, wr, b, out):
  mean = (agg0[0] + agg1[0]) * inv[...]
  z = (jnp.dot(mean, wl[...], preferred_element_type=jnp.float32,
               precision=lax.Precision.HIGHEST)
       + jnp.dot(h[...], wr[...], preferred_element_type=jnp.float32,
                 precision=lax.Precision.HIGHEST) + b[...])
  if last:
    m = jnp.max(z, axis=-1, keepdims=True)
    lse = jnp.log(jnp.sum(jnp.exp(z - m), axis=-1, keepdims=True)) + m
    out[...] = z - lse
  else:
    out[...] = _elu(z)


_agg_spec = lambda core: pl.BlockSpec((1, ROWS_BLK, D), lambda i, c=core: (c, i, 0))
_cnt_spec = lambda core: pl.BlockSpec((1, ROWS_BLK, CNTW), lambda i, c=core: (c, i, 0))
_row_spec = pl.BlockSpec((ROWS_BLK, D), lambda i: (i, 0))
_w_spec = pl.BlockSpec((D, D), lambda i: (0, 0))
_b_spec = pl.BlockSpec((1, D), lambda i: (0, 0))
_GRID = (N // ROWS_BLK,)

_tc_first = pl.pallas_call(
    _tc_first_body,
    grid=_GRID,
    in_specs=[_agg_spec(0), _agg_spec(1), _cnt_spec(0), _cnt_spec(1),
              _row_spec, _w_spec, _w_spec, _b_spec],
    out_specs=[_row_spec, _row_spec],
    out_shape=[jax.ShapeDtypeStruct((N, D), jnp.float32),
               jax.ShapeDtypeStruct((N, D), jnp.float32)],
)

_tc_mid = pl.pallas_call(
    functools.partial(_tc_mid_body, False),
    grid=_GRID,
    in_specs=[_agg_spec(0), _agg_spec(1), _row_spec,
              _row_spec, _w_spec, _w_spec, _b_spec],
    out_specs=_row_spec,
    out_shape=jax.ShapeDtypeStruct((N, D), jnp.float32),
)

_tc_last = pl.pallas_call(
    functools.partial(_tc_mid_body, True),
    grid=_GRID,
    in_specs=[_agg_spec(0), _agg_spec(1), _row_spec,
              _row_spec, _w_spec, _w_spec, _b_spec],
    out_specs=_row_spec,
    out_shape=jax.ShapeDtypeStruct((N, D), jnp.float32),
)


def kernel(x, edge_index, W1l, W1r, b1, W2l, W2r, b2, W3l, W3r, b3,
           W4l, W4r, b4):
  src = edge_index[0].astype(jnp.int32)
  dst = edge_index[1].astype(jnp.int32)
  npad_e = EPAD - E
  src_p = jnp.concatenate([src, jnp.zeros((npad_e,), jnp.int32)])
  dst_p = jnp.concatenate(
      [dst, N + (jnp.arange(npad_e, dtype=jnp.int32) % CNTW)])
  z128 = jnp.zeros((NPAD, D), jnp.float32)
  z16 = jnp.zeros((NPAD, CNTW), jnp.float32)
  ones16 = jnp.ones((CHUNK, CNTW), jnp.float32)

  agg1, cnt = _make_sc_agg(True)(x, src_p, dst_p, z128, z16, ones16)
  _sc_agg = _make_sc_agg(False)
  h1, inv = _tc_first(agg1, agg1, cnt, cnt, x,
                      W1l, W1r, b1.reshape(1, D))
  agg2 = _sc_agg(h1, src_p, dst_p, z128)
  h2 = _tc_mid(agg2, agg2, inv, h1, W2l, W2r, b2.reshape(1, D))
  agg3 = _sc_agg(h2, src_p, dst_p, z128)
  h3 = _tc_mid(agg3, agg3, inv, h2, W3l, W3r, b3.reshape(1, D))
  agg4 = _sc_agg(h3, src_p, dst_p, z128)

  dout = W4l.shape[1]
  W4l_p = jnp.zeros((D, D), jnp.float32).at[:, :dout].set(W4l)
  W4r_p = jnp.zeros((D, D), jnp.float32).at[:, :dout].set(W4r)
  b4_p = jnp.full((1, D), -1e30, jnp.float32).at[0, :dout].set(b4)
  out = _tc_last(agg4, agg4, inv, h3, W4l_p, W4r_p, b4_p)
  return out[:, :dout]
